# baseline (speedup 1.0000x reference)
import functools

import numpy as np

import jax
import jax.numpy as jnp
from jax.experimental import pallas as pl
from jax.experimental.pallas import tpu as pltpu


_VMEM_LIMIT = 48 * 1024 * 1024
_IMGS_PER_STEP = 8


def _dsconv_kernel(x_ref, s_ref, pw_ref, be_ref, m_ref, o_ref, xb,
                   *, W, L, F, PAD, LE):
    B = x_ref.shape[0]
    C = x_ref.shape[1]
    XB = xb.shape[2]
    ml = m_ref[0:1, :]
    mr = m_ref[1:2, :]

    def sc(kh, kw):
        j = kh * 3 + kw
        return s_ref[:, j:j + 1]

    @pl.when(pl.program_id(0) == 0)
    def _zero_halos():
        xb[...] = jnp.zeros(xb.shape, jnp.bfloat16)

    for i in range(B):
        b = xb.at[i % 2]
        b[:, PAD:PAD + L] = x_ref[i]

        u0 = b[:, PAD - F - W:PAD - F - W + LE]
        u1 = b[:, PAD - F:PAD - F + LE]
        u2 = b[:, PAD - F + W:PAD - F + W + LE]

        tl = sc(0, 0) * u0 + sc(1, 0) * u1 + sc(2, 0) * u2
        tc = sc(0, 1) * u0 + sc(1, 1) * u1 + sc(2, 1) * u2
        tr = sc(0, 2) * u0 + sc(1, 2) * u1 + sc(2, 2) * u2

        z = (tc[:, F:F + L]
             + ml * tl[:, F - 1:F - 1 + L]
             + mr * tr[:, F + 1:F + 1 + L])

        y = jnp.dot(pw_ref[...], z, preferred_element_type=jnp.float32)
        o_ref[i] = (y + be_ref[...]).astype(o_ref.dtype)


def kernel(x_nchw, dw_w, dw_b, pw_w, pw_b):
    N, C, H, W = x_nchw.shape
    KH, KW = dw_w.shape[2], dw_w.shape[3]
    O = pw_w.shape[0]
    L = H * W
    F = 128
    PAD = 2 * F
    assert F >= W + 1
    LE = -(-(F + L + 1) // 128) * 128
    XB = -(-(PAD - F + W + LE) // 128) * 128
    B = _IMGS_PER_STEP if N % _IMGS_PER_STEP == 0 else 1

    s = dw_w[:, 0, :, :].reshape(C, KH * KW).astype(jnp.bfloat16)
    pw_mat = pw_w[:, :, 0, 0].astype(jnp.float32)
    pw = pw_mat.astype(jnp.bfloat16)
    be = (pw_b.astype(jnp.float32)
          + pw_mat @ dw_b.astype(jnp.float32)).reshape(O, 1)

    w_idx = np.arange(L, dtype=np.int64) % W
    masks_np = np.zeros((16, L), np.float32)
    masks_np[0] = (w_idx != 0).astype(np.float32)
    masks_np[1] = (w_idx != W - 1).astype(np.float32)
    masks = jnp.asarray(masks_np, dtype=jnp.bfloat16)

    x_flat = x_nchw.reshape(N, C, L).astype(jnp.bfloat16)
    kern = functools.partial(_dsconv_kernel, W=W, L=L, F=F, PAD=PAD, LE=LE)
    out = pl.pallas_call(
        kern,
        out_shape=jax.ShapeDtypeStruct((N, O, L), jnp.bfloat16),
        grid=(N // B,),
        in_specs=[
            pl.BlockSpec((B, C, L), lambda n: (n, 0, 0)),
            pl.BlockSpec((C, KH * KW), lambda n: (0, 0)),
            pl.BlockSpec((O, C), lambda n: (0, 0)),
            pl.BlockSpec((O, 1), lambda n: (0, 0)),
            pl.BlockSpec((16, L), lambda n: (0, 0)),
        ],
        out_specs=pl.BlockSpec((B, O, L), lambda n: (n, 0, 0)),
        scratch_shapes=[pltpu.VMEM((2, C, XB), jnp.bfloat16)],
        compiler_params=pltpu.CompilerParams(
            dimension_semantics=("arbitrary",),
            vmem_limit_bytes=_VMEM_LIMIT),
    )(x_flat, s, pw, be, masks)

    return out.reshape(N, O, H, W).astype(x_nchw.dtype)


# speedup vs baseline: 1.0021x; 1.0021x over previous
import functools

import numpy as np

import jax
import jax.numpy as jnp
from jax.experimental import pallas as pl
from jax.experimental.pallas import tpu as pltpu


_VMEM_LIMIT = 48 * 1024 * 1024
_IMGS_PER_STEP = 8


def _dsconv_kernel(x_ref, s_ref, pw_ref, be_ref, m_ref, o_ref, xb,
                   *, W, L, F, PAD, LE):
    B = x_ref.shape[0]
    C = x_ref.shape[1]
    XB = xb.shape[2]
    ml = m_ref[0:1, :]
    mr = m_ref[1:2, :]

    def sc(kh, kw):
        j = kh * 3 + kw
        return s_ref[:, j:j + 1]

    @pl.when(pl.program_id(0) == 0)
    def _zero_halos():
        xb[...] = jnp.zeros(xb.shape, jnp.bfloat16)

    for i in range(B):
        b = xb.at[i % 2]
        b[:, PAD:PAD + L] = x_ref[i]

        u0 = b[:, PAD - F - W:PAD - F - W + LE]
        u1 = b[:, PAD - F:PAD - F + LE]
        u2 = b[:, PAD - F + W:PAD - F + W + LE]

        tl = sc(0, 0) * u0 + sc(1, 0) * u1 + sc(2, 0) * u2
        tc = sc(0, 1) * u0 + sc(1, 1) * u1 + sc(2, 1) * u2
        tr = sc(0, 2) * u0 + sc(1, 2) * u1 + sc(2, 2) * u2

        z = (tc[:, F:F + L]
             + ml * tl[:, F - 1:F - 1 + L]
             + mr * tr[:, F + 1:F + 1 + L])

        y = jnp.dot(pw_ref[...], z, preferred_element_type=jnp.float32)
        o_ref[i] = (y + be_ref[...]).astype(o_ref.dtype)


def kernel(x_nchw, dw_w, dw_b, pw_w, pw_b):
    N, C, H, W = x_nchw.shape
    KH, KW = dw_w.shape[2], dw_w.shape[3]
    O = pw_w.shape[0]
    L = H * W
    F = 128
    PAD = 2 * F
    assert F >= W + 1
    LE = -(-(F + L + 1) // 128) * 128
    XB = -(-(PAD - F + W + LE) // 128) * 128
    B = _IMGS_PER_STEP if N % _IMGS_PER_STEP == 0 else 1

    s = dw_w[:, 0, :, :].reshape(C, KH * KW).astype(jnp.bfloat16)
    pw_mat = pw_w[:, :, 0, 0].astype(jnp.float32)
    pw = pw_mat.astype(jnp.bfloat16)
    be = (pw_b.astype(jnp.float32)
          + pw_mat @ dw_b.astype(jnp.float32)).reshape(O, 1)

    w_idx = np.arange(L, dtype=np.int64) % W
    masks_np = np.zeros((16, L), np.float32)
    masks_np[0] = (w_idx != 0).astype(np.float32)
    masks_np[1] = (w_idx != W - 1).astype(np.float32)
    masks = jnp.asarray(masks_np, dtype=jnp.bfloat16)

    x_flat = x_nchw.reshape(N, C, L).astype(jnp.bfloat16)
    kern = functools.partial(_dsconv_kernel, W=W, L=L, F=F, PAD=PAD, LE=LE)
    out = pl.pallas_call(
        kern,
        out_shape=jax.ShapeDtypeStruct((N, O, L), jnp.bfloat16),
        grid=(N // B,),
        in_specs=[
            pl.BlockSpec((B, C, L), lambda n: (n, 0, 0)),
            pl.BlockSpec((C, KH * KW), lambda n: (0, 0)),
            pl.BlockSpec((O, C), lambda n: (0, 0)),
            pl.BlockSpec((O, 1), lambda n: (0, 0)),
            pl.BlockSpec((16, L), lambda n: (0, 0)),
        ],
        out_specs=pl.BlockSpec((B, O, L), lambda n: (n, 0, 0)),
        scratch_shapes=[pltpu.VMEM((2, C, XB), jnp.bfloat16)],
        compiler_params=pltpu.CompilerParams(
            dimension_semantics=("arbitrary",),
            allow_input_fusion=[True, False, False, False, False],
            vmem_limit_bytes=_VMEM_LIMIT),
    )(x_flat, s, pw, be, masks)

    return out.reshape(N, O, H, W).astype(x_nchw.dtype)


# speedup vs baseline: 1.3252x; 1.3224x over previous
import functools

import numpy as np

import jax
import jax.numpy as jnp
from jax.experimental import pallas as pl
from jax.experimental.pallas import tpu as pltpu


_VMEM_LIMIT = 48 * 1024 * 1024
_IMGS_PER_STEP = 8


def _dsconv_kernel(x_ref, s_ref, pw_ref, be_ref, m_ref, o_ref, xb,
                   *, W, L, F, PAD, LE):
    B = x_ref.shape[0]
    C = x_ref.shape[1]
    XB = xb.shape[2]
    ml = m_ref[0:1, :]
    mr = m_ref[1:2, :]

    def sc(kh, kw):
        j = kh * 3 + kw
        return s_ref[:, j:j + 1]

    zfront = jnp.zeros((C, PAD), jnp.bfloat16)
    zback = jnp.zeros((C, XB - PAD - L), jnp.bfloat16)

    for i in range(B):
        ext = jnp.concatenate([zfront, x_ref[i], zback], axis=1)

        u0 = ext[:, PAD - F - W:PAD - F - W + LE]
        u1 = ext[:, PAD - F:PAD - F + LE]
        u2 = ext[:, PAD - F + W:PAD - F + W + LE]

        tl = sc(0, 0) * u0 + sc(1, 0) * u1 + sc(2, 0) * u2
        tc = sc(0, 1) * u0 + sc(1, 1) * u1 + sc(2, 1) * u2
        tr = sc(0, 2) * u0 + sc(1, 2) * u1 + sc(2, 2) * u2

        z = (tc[:, F:F + L]
             + ml * tl[:, F - 1:F - 1 + L]
             + mr * tr[:, F + 1:F + 1 + L])

        y = jnp.dot(pw_ref[...], z, preferred_element_type=jnp.float32)
        o_ref[i] = (y + be_ref[...]).astype(o_ref.dtype)


def kernel(x_nchw, dw_w, dw_b, pw_w, pw_b):
    N, C, H, W = x_nchw.shape
    KH, KW = dw_w.shape[2], dw_w.shape[3]
    O = pw_w.shape[0]
    L = H * W
    F = 128
    PAD = 2 * F
    assert F >= W + 1
    LE = -(-(F + L + 1) // 128) * 128
    XB = -(-(PAD - F + W + LE) // 128) * 128
    B = _IMGS_PER_STEP if N % _IMGS_PER_STEP == 0 else 1

    s = dw_w[:, 0, :, :].reshape(C, KH * KW).astype(jnp.bfloat16)
    pw_mat = pw_w[:, :, 0, 0].astype(jnp.float32)
    pw = pw_mat.astype(jnp.bfloat16)
    be = (pw_b.astype(jnp.float32)
          + pw_mat @ dw_b.astype(jnp.float32)).reshape(O, 1)

    w_idx = np.arange(L, dtype=np.int64) % W
    masks_np = np.zeros((16, L), np.float32)
    masks_np[0] = (w_idx != 0).astype(np.float32)
    masks_np[1] = (w_idx != W - 1).astype(np.float32)
    masks = jnp.asarray(masks_np, dtype=jnp.bfloat16)

    x_flat = x_nchw.reshape(N, C, L).astype(jnp.bfloat16)
    kern = functools.partial(_dsconv_kernel, W=W, L=L, F=F, PAD=PAD, LE=LE)
    out = pl.pallas_call(
        kern,
        out_shape=jax.ShapeDtypeStruct((N, O, L), jnp.bfloat16),
        grid=(N // B,),
        in_specs=[
            pl.BlockSpec((B, C, L), lambda n: (n, 0, 0)),
            pl.BlockSpec((C, KH * KW), lambda n: (0, 0)),
            pl.BlockSpec((O, C), lambda n: (0, 0)),
            pl.BlockSpec((O, 1), lambda n: (0, 0)),
            pl.BlockSpec((16, L), lambda n: (0, 0)),
        ],
        out_specs=pl.BlockSpec((B, O, L), lambda n: (n, 0, 0)),
        scratch_shapes=[pltpu.VMEM((2, C, XB), jnp.bfloat16)],
        compiler_params=pltpu.CompilerParams(
            dimension_semantics=("arbitrary",),
            allow_input_fusion=[True, False, False, False, False],
            vmem_limit_bytes=_VMEM_LIMIT),
    )(x_flat, s, pw, be, masks)

    return out.reshape(N, O, H, W).astype(x_nchw.dtype)
